# baseline (speedup 1.0000x reference)
import functools

import jax
import jax.numpy as jnp
from jax import lax
from jax.experimental import pallas as pl
from jax.experimental.pallas import tpu as pltpu
from jax.experimental.pallas import tpu_sc as plsc

_F = 64
_CHUNK = 128


@functools.lru_cache(maxsize=None)
def _make_gather(batch, num_rows):
    info = plsc.get_sparse_core_info()
    nc, ns = info.num_cores, info.num_subcores
    nw = nc * ns
    bpw = batch // nw
    nch = bpw // _CHUNK
    mesh = plsc.VectorSubcoreMesh(core_axis_name="c", subcore_axis_name="s")

    @functools.partial(
        pl.kernel,
        mesh=mesh,
        out_type=[
            jax.ShapeDtypeStruct((batch, _F), jnp.float32),
            jax.ShapeDtypeStruct((batch, _F), jnp.float32),
        ],
        scratch_types=[
            pltpu.VMEM((nch, _CHUNK), jnp.int32),
            pltpu.VMEM((nch, _CHUNK), jnp.int32),
            pltpu.VMEM((bpw, _F), jnp.float32),
            pltpu.VMEM((bpw, _F), jnp.float32),
            pltpu.SemaphoreType.DMA,
        ],
        compiler_params=pltpu.CompilerParams(use_tc_tiling_on_sc=False),
    )
    def gather_k(ti_hbm, ci_hbm, tt_hbm, ct_hbm, out_t, out_c,
                 ti_v, ci_v, rt_v, rc_v, sem):
        wid = lax.axis_index("s") * nc + lax.axis_index("c")
        base = wid * bpw
        pltpu.sync_copy(ti_hbm.at[wid], ti_v)
        pltpu.sync_copy(ci_hbm.at[wid], ci_v)
        copies = []
        for c in range(nch):
            dst = pl.ds(c * _CHUNK, _CHUNK)
            copies.append(pltpu.async_copy(tt_hbm.at[ti_v.at[c]], rt_v.at[dst], sem))
            copies.append(pltpu.async_copy(ct_hbm.at[ci_v.at[c]], rc_v.at[dst], sem))
        for cp in copies:
            cp.wait()
        pltpu.sync_copy(rt_v, out_t.at[pl.ds(base, bpw)])
        pltpu.sync_copy(rc_v, out_c.at[pl.ds(base, bpw)])

    return gather_k, nw, nch


def _mlp_body(te_ref, ce_ref, w1a_ref, w1b_ref, b1_ref, w2_ref, b2_ref,
              w3t_ref, b3_ref, out_ref):
    h = jnp.dot(te_ref[...], w1a_ref[...], preferred_element_type=jnp.float32)
    h = h + jnp.dot(ce_ref[...], w1b_ref[...], preferred_element_type=jnp.float32)
    h = jnp.maximum(h + b1_ref[...], 0.0)
    h = jnp.dot(h, w2_ref[...], preferred_element_type=jnp.float32)
    h = jnp.maximum(h + b2_ref[...], 0.0)
    o = jnp.sum(h * w3t_ref[...], axis=1, keepdims=True) + b3_ref[...]
    out_ref[...] = jax.nn.sigmoid(o)


@functools.lru_cache(maxsize=None)
def _make_mlp(batch, bb):
    grid = (batch // bb,)
    full = lambda shape: pl.BlockSpec(shape, lambda i: (0, 0))
    return pl.pallas_call(
        _mlp_body,
        grid=grid,
        in_specs=[
            pl.BlockSpec((bb, _F), lambda i: (i, 0)),
            pl.BlockSpec((bb, _F), lambda i: (i, 0)),
            full((_F, 128)),
            full((_F, 128)),
            full((1, 128)),
            full((128, _F)),
            full((1, _F)),
            full((1, _F)),
            full((1, 1)),
        ],
        out_specs=pl.BlockSpec((bb, 1), lambda i: (i, 0)),
        out_shape=jax.ShapeDtypeStruct((batch, 1), jnp.float32),
    )


def kernel(track_indices, context_indices, track_table, context_table,
           W1, b1, W2, b2, W3, b3):
    batch = track_indices.shape[0]
    gather_k, nw, nch = _make_gather(batch, track_table.shape[0])
    ti = track_indices.astype(jnp.int32).reshape(nw, nch, _CHUNK)
    ci = context_indices.astype(jnp.int32).reshape(nw, nch, _CHUNK)
    te, ce = gather_k(ti, ci, track_table, context_table)
    mlp = _make_mlp(batch, 2048)
    return mlp(te, ce, W1[:_F], W1[_F:], b1.reshape(1, 128), W2,
               b2.reshape(1, _F), W3.reshape(1, _F), b3.reshape(1, 1))
